# baseline (speedup 1.0000x reference)
import functools
import jax
import jax.numpy as jnp
from jax import lax
from jax.experimental import pallas as pl
from jax.experimental.pallas import tpu as pltpu
from jax.experimental.pallas import tpu_sc as plsc

N = 10000
E = 320000
C = 128
D = 32
NM = 13
NNZ = 320000
H = 64
RB = 1000
NRB = N // RB


def _dense_a_kernel(x_ref, w_ref, r_ref, b_ref, xw_ref, base_ref):
    xb = x_ref[...]
    xw_ref[0] = jnp.dot(xb, w_ref[0], preferred_element_type=jnp.float32)
    base_ref[0] = jnp.dot(xb, r_ref[0], preferred_element_type=jnp.float32) + b_ref[0]


def _dense_a(x, W_conv, root, bias):
    w_split = W_conv.reshape(C, 2, H).transpose(1, 0, 2)
    r_split = root.reshape(C, 2, H).transpose(1, 0, 2)
    b_split = bias.reshape(2, 1, H)
    return pl.pallas_call(
        _dense_a_kernel,
        grid=(2, NRB),
        in_specs=[
            pl.BlockSpec((RB, C), lambda c, i: (i, 0)),
            pl.BlockSpec((1, C, H), lambda c, i: (c, 0, 0)),
            pl.BlockSpec((1, C, H), lambda c, i: (c, 0, 0)),
            pl.BlockSpec((1, 1, H), lambda c, i: (c, 0, 0)),
        ],
        out_specs=[
            pl.BlockSpec((1, RB, H), lambda c, i: (c, i, 0)),
            pl.BlockSpec((1, RB, H), lambda c, i: (c, i, 0)),
        ],
        out_shape=[
            jax.ShapeDtypeStruct((2, NP, H), jnp.float32),
            jax.ShapeDtypeStruct((2, NP, H), jnp.float32),
        ],
    )(x, w_split, r_split, b_split)


def _attn_kernel(h_ref, s_ref, vc_ref, cb_ref, mb_ref, o_ref):
    acc = jnp.zeros((RB, 2 * NM * D), jnp.float32)
    for m in range(NM + 1):
        if m == 0:
            rm = jnp.concatenate([h_ref[0], h_ref[1]], axis=1)
        else:
            rm = jnp.concatenate([s_ref[m - 1, 0], s_ref[m - 1, 1]], axis=1)
        acc = acc + jnp.dot(rm, vc_ref[m], preferred_element_type=jnp.float32)
    zc = acc[:, : NM * D] + cb_ref[...]
    zm = acc[:, NM * D :] + mb_ref[...]
    g = (lax.broadcasted_iota(jnp.int32, (NM * D, NM), 0) // D
         == lax.broadcasted_iota(jnp.int32, (NM * D, NM), 1)).astype(jnp.float32)
    logits = jnp.dot(zc * zm, g, preferred_element_type=jnp.float32)
    att = jax.nn.sigmoid(logits)
    att_e = jnp.dot(att, g.T, preferred_element_type=jnp.float32)
    o_ref[...] = att_e * (zm - zc)


def _attn(h_pair, s_all, vc, cb, mb):
    return pl.pallas_call(
        _attn_kernel,
        grid=(NRB,),
        in_specs=[
            pl.BlockSpec((2, RB, H), lambda i: (0, i, 0)),
            pl.BlockSpec((NM, 2, RB, H), lambda i: (0, 0, i, 0)),
            pl.BlockSpec((NM + 1, C, 2 * NM * D), lambda i: (0, 0, 0)),
            pl.BlockSpec((1, NM * D), lambda i: (0, 0)),
            pl.BlockSpec((1, NM * D), lambda i: (0, 0)),
        ],
        out_specs=pl.BlockSpec((RB, NM * D), lambda i: (i, 0)),
        out_shape=jax.ShapeDtypeStruct((N, NM * D), jnp.float32),
    )(h_pair, s_all, vc, cb, mb)


NS = 16
NP = 10240
RT = NP // NS
K = 128
CH = E // (NS * K)
EBASE = NS * CH * K
NQ = 32


def _sc_body(xw_hbm, base_hbm, ep_hbm, mp_hbm,
             h_out, s_out,
             idx3, grows2, nbuf, bbuf, hbuf, zbuf, dub, degbuf,
             acc, deg2d, semi, semg, sems):
    c = lax.axis_index("c")
    s = lax.axis_index("s")
    r0 = s * RT
    nchunk = jnp.where(s < 4, CH + 1, CH)

    z16 = jnp.zeros((16,), jnp.float32)
    dconst = jnp.where(jnp.arange(16, dtype=jnp.int32) == 0, 1.0, 0.0).astype(jnp.float32)

    def zrow(j, _):
        for f in range(H // 16):
            zbuf[j, pl.ds(f * 16, 16)] = z16
        degbuf[j, pl.ds(0, 16)] = z16
        return 0
    lax.fori_loop(0, NQ, zrow, 0)
    def drow(j, _):
        dub[j, pl.ds(0, 16)] = dconst
        return 0
    lax.fori_loop(0, K, drow, 0)
    def zacc(q, _):
        pltpu.sync_copy(zbuf, acc.at[pl.ds(r0 + q * NQ, NQ)])
        pltpu.sync_copy(degbuf, deg2d.at[pl.ds(r0 + q * NQ, NQ)])
        return 0
    lax.fori_loop(0, RT // NQ, zacc, 0)
    plsc.subcore_barrier()

    def run_pipeline(e0_of, psrc, gtab, with_deg):
        def load(g, b):
            e0 = e0_of(g)
            pltpu.async_copy(psrc.at[:, pl.ds(e0, K)], idx3.at[b], semi)
        def wait_load(b):
            pltpu.make_async_copy(psrc.at[:, pl.ds(0, K)], idx3.at[b], semi).wait()
        def gather_start(b):
            pltpu.async_copy(gtab.at[idx3.at[b, 0]], grows2.at[b], semg.at[b])
        def wait_gather(b):
            pltpu.make_async_copy(gtab.at[pl.ds(0, K)], grows2.at[b], semg.at[b]).wait()
        def scatter_start(b):
            pltpu.async_copy(grows2.at[b], acc.at[idx3.at[b, 1]], sems.at[b], add=True)
            if with_deg:
                pltpu.async_copy(dub, deg2d.at[idx3.at[b, 1]], sems.at[b], add=True)
        def wait_scatter(b):
            pltpu.make_async_copy(grows2.at[b], acc.at[pl.ds(0, K)], sems.at[b]).wait()
            if with_deg:
                pltpu.make_async_copy(dub, deg2d.at[pl.ds(0, K)], sems.at[b]).wait()
        def scale(b):
            @plsc.parallel_loop(0, K // 16, unroll=K // 16)
            def blk(t):
                j0 = t * 16
                vv = plsc.bitcast(idx3[b, 2, pl.ds(j0, 16)], jnp.float32)
                for jj in range(16):
                    v = jnp.full((16,), vv[jj], jnp.float32)
                    for f in range(H // 16):
                        sl = pl.ds(f * 16, 16)
                        grows2[b, j0 + jj, sl] = grows2[b, j0 + jj, sl] * v

        load(0, 0)
        wait_load(0)
        gather_start(0)
        load(1, 1)
        def body(g, _):
            b = jnp.bitwise_and(g, 1)
            nb = 1 - b
            @pl.when(g + 1 < nchunk)
            def _():
                wait_load(nb)
                @pl.when(g >= 1)
                def _():
                    wait_scatter(nb)
                gather_start(nb)
            wait_gather(b)
            scale(b)
            scatter_start(b)
            @pl.when(g + 2 < nchunk)
            def _():
                load(g + 2, b)
            return 0
        lax.fori_loop(0, nchunk, body, 0)
        wait_scatter(0)
        wait_scatter(1)

    def chunk_off(g):
        return jnp.where(g < CH, s * (CH * K) + g * K, EBASE + s * K)

    run_pipeline(chunk_off, ep_hbm, xw_hbm.at[c], True)
    plsc.subcore_barrier()

    def norm_q(q, _):
        r = r0 + q * NQ
        pltpu.sync_copy(acc.at[pl.ds(r, NQ)], nbuf)
        pltpu.sync_copy(zbuf, acc.at[pl.ds(r, NQ)])
        pltpu.sync_copy(base_hbm.at[c, pl.ds(r, NQ)], bbuf)
        pltpu.sync_copy(deg2d.at[pl.ds(r, NQ)], degbuf)
        def nrow(j, _):
            dv = degbuf[j, pl.ds(0, 16)]
            ivv = jnp.where(dv > 0.0, 1.0 / dv, 0.0)
            iv = jnp.full((16,), ivv[0], jnp.float32)
            for f in range(H // 16):
                sl = pl.ds(f * 16, 16)
                hbuf[j, sl] = nbuf[j, sl] * iv + bbuf[j, sl]
            return 0
        lax.fori_loop(0, NQ, nrow, 0)
        pltpu.sync_copy(hbuf, h_out.at[c, pl.ds(r, NQ)])
        return 0
    lax.fori_loop(0, RT // NQ, norm_q, 0)
    plsc.subcore_barrier()

    def motif_body(i, _):
        run_pipeline(lambda g: i * NNZ + chunk_off(g), mp_hbm, h_out.at[c], False)
        plsc.subcore_barrier()
        pltpu.sync_copy(acc.at[pl.ds(r0, RT)], s_out.at[i, c, pl.ds(r0, RT)])
        def zacc2(q, __):
            pltpu.sync_copy(zbuf, acc.at[pl.ds(r0 + q * NQ, NQ)])
            return 0
        lax.fori_loop(0, RT // NQ, zacc2, 0)
        plsc.subcore_barrier()
        return 0
    lax.fori_loop(0, NM, motif_body, 0)


def _sc_sparse(xw_pair, base_pair, epack, mpack):
    mesh = plsc.VectorSubcoreMesh(core_axis_name="c", subcore_axis_name="s")
    f = pl.kernel(
        _sc_body,
        out_type=[
            jax.ShapeDtypeStruct((2, NP, H), jnp.float32),
            jax.ShapeDtypeStruct((NM, 2, NP, H), jnp.float32),
        ],
        mesh=mesh,
        compiler_params=pltpu.CompilerParams(use_tc_tiling_on_sc=False,
                                             needs_layout_passes=False),
        scratch_types=[
            pltpu.VMEM((2, 3, K), jnp.int32),
            pltpu.VMEM((2, K, H), jnp.float32),
            pltpu.VMEM((NQ, H), jnp.float32),
            pltpu.VMEM((NQ, H), jnp.float32),
            pltpu.VMEM((NQ, H), jnp.float32),
            pltpu.VMEM((NQ, H), jnp.float32),
            pltpu.VMEM((K, 16), jnp.float32),
            pltpu.VMEM((NQ, 16), jnp.float32),
            pltpu.VMEM_SHARED((NP, H), jnp.float32),
            pltpu.VMEM_SHARED((NP, 16), jnp.float32),
            pltpu.SemaphoreType.DMA,
            pltpu.SemaphoreType.DMA((2,)),
            pltpu.SemaphoreType.DMA((2,)),
        ],
    )
    return f(xw_pair, base_pair, epack, mpack)


def _build_combined_weights(wa, motif_w):
    blocks = motif_w.reshape(NM, NM, C, D)
    vc = jnp.zeros((NM + 1, C, 2 * NM * D), jnp.float32)
    for i in range(1, NM + 1):
        for j in range(NM + 1):
            if j == i:
                continue
            jj = j if j < i else j - 1
            vc = vc.at[j, :, (i - 1) * D : i * D].set(blocks[i - 1, jj])
        vc = vc.at[i, :, NM * D + (i - 1) * D : NM * D + i * D].set(wa)
    return vc


def kernel(x, edge_weight, motif_val, W_conv, root, bias, wa, ba, motif_w, motif_b, edge_index, motif_idx):
    xw_pair, base_pair = _dense_a(x, W_conv, root, bias)

    epack = jnp.stack([
        edge_index[0], edge_index[1],
        lax.bitcast_convert_type(edge_weight, jnp.int32),
    ])
    mpack = jnp.stack([
        motif_idx[:, 1].reshape(-1), motif_idx[:, 0].reshape(-1),
        lax.bitcast_convert_type(motif_val.reshape(-1), jnp.int32),
    ])
    h_pair, s_all = _sc_sparse(xw_pair, base_pair, epack, mpack)

    vc = _build_combined_weights(wa, motif_w)
    cb = motif_b.reshape(1, NM * D)
    mb = jnp.tile(ba, NM).reshape(1, NM * D)
    return _attn(h_pair, s_all, vc, cb, mb)


# speedup vs baseline: 1.2263x; 1.2263x over previous
import functools
import jax
import jax.numpy as jnp
from jax import lax
from jax.experimental import pallas as pl
from jax.experimental.pallas import tpu as pltpu
from jax.experimental.pallas import tpu_sc as plsc

N = 10000
E = 320000
C = 128
D = 32
NM = 13
NNZ = 320000
H = 64
RB = 1000
NRB = N // RB


def _dense_a_kernel(x_ref, w_ref, r_ref, b_ref, xw_ref, base_ref):
    xb = x_ref[...]
    xw_ref[0] = jnp.dot(xb, w_ref[0], preferred_element_type=jnp.float32)
    base_ref[0] = jnp.dot(xb, r_ref[0], preferred_element_type=jnp.float32) + b_ref[0]


def _dense_a(x, W_conv, root, bias):
    w_split = W_conv.reshape(C, 2, H).transpose(1, 0, 2)
    r_split = root.reshape(C, 2, H).transpose(1, 0, 2)
    b_split = bias.reshape(2, 1, H)
    return pl.pallas_call(
        _dense_a_kernel,
        grid=(2, NRB),
        in_specs=[
            pl.BlockSpec((RB, C), lambda c, i: (i, 0)),
            pl.BlockSpec((1, C, H), lambda c, i: (c, 0, 0)),
            pl.BlockSpec((1, C, H), lambda c, i: (c, 0, 0)),
            pl.BlockSpec((1, 1, H), lambda c, i: (c, 0, 0)),
        ],
        out_specs=[
            pl.BlockSpec((1, RB, H), lambda c, i: (c, i, 0)),
            pl.BlockSpec((1, RB, H), lambda c, i: (c, i, 0)),
        ],
        out_shape=[
            jax.ShapeDtypeStruct((2, NP, H), jnp.float32),
            jax.ShapeDtypeStruct((2, NP, H), jnp.float32),
        ],
    )(x, w_split, r_split, b_split)


def _attn_kernel(h_ref, s_ref, vc_ref, cb_ref, mb_ref, o_ref):
    acc = jnp.zeros((RB, 2 * NM * D), jnp.float32)
    for m in range(NM + 1):
        if m == 0:
            rm = jnp.concatenate([h_ref[0], h_ref[1]], axis=1)
        else:
            rm = jnp.concatenate([s_ref[m - 1, 0], s_ref[m - 1, 1]], axis=1)
        acc = acc + jnp.dot(rm, vc_ref[m], preferred_element_type=jnp.float32)
    zc = acc[:, : NM * D] + cb_ref[...]
    zm = acc[:, NM * D :] + mb_ref[...]
    g = (lax.broadcasted_iota(jnp.int32, (NM * D, NM), 0) // D
         == lax.broadcasted_iota(jnp.int32, (NM * D, NM), 1)).astype(jnp.float32)
    logits = jnp.dot(zc * zm, g, preferred_element_type=jnp.float32)
    att = jax.nn.sigmoid(logits)
    att_e = jnp.dot(att, g.T, preferred_element_type=jnp.float32)
    o_ref[...] = att_e * (zm - zc)


def _attn(h_pair, s_all, vc, cb, mb):
    return pl.pallas_call(
        _attn_kernel,
        grid=(NRB,),
        in_specs=[
            pl.BlockSpec((2, RB, H), lambda i: (0, i, 0)),
            pl.BlockSpec((NM, 2, RB, H), lambda i: (0, 0, i, 0)),
            pl.BlockSpec((NM + 1, C, 2 * NM * D), lambda i: (0, 0, 0)),
            pl.BlockSpec((1, NM * D), lambda i: (0, 0)),
            pl.BlockSpec((1, NM * D), lambda i: (0, 0)),
        ],
        out_specs=pl.BlockSpec((RB, NM * D), lambda i: (i, 0)),
        out_shape=jax.ShapeDtypeStruct((N, NM * D), jnp.float32),
    )(h_pair, s_all, vc, cb, mb)


NS = 16
NP = 10240
RT = NP // NS
K = 128
CH = E // (NS * K)
EBASE = NS * CH * K
NQ = 32


def _sc_body(xw_hbm, base_hbm, ep_hbm, mp_hbm,
             h_out, s_out,
             idx3, grows2, nbuf, bbuf, hbuf, zbuf, dub, degbuf,
             acc, deg2d, semi, semg, sems):
    c = lax.axis_index("c")
    s = lax.axis_index("s")
    r0 = s * RT
    nchunk = jnp.where(s < 4, CH + 1, CH)

    z16 = jnp.zeros((16,), jnp.float32)
    dconst = jnp.where(jnp.arange(16, dtype=jnp.int32) == 0, 1.0, 0.0).astype(jnp.float32)

    def zrow(j, _):
        for f in range(H // 16):
            zbuf[j, pl.ds(f * 16, 16)] = z16
        degbuf[j, pl.ds(0, 16)] = z16
        return 0
    lax.fori_loop(0, NQ, zrow, 0)
    def drow(j, _):
        dub[j, pl.ds(0, 16)] = dconst
        return 0
    lax.fori_loop(0, K, drow, 0)
    def zacc(q, _):
        pltpu.sync_copy(zbuf, acc.at[pl.ds(r0 + q * NQ, NQ)])
        pltpu.sync_copy(degbuf, deg2d.at[pl.ds(r0 + q * NQ, NQ)])
        return 0
    lax.fori_loop(0, RT // NQ, zacc, 0)
    plsc.subcore_barrier()

    def run_pipeline(e0_of, psrc, gtab, with_deg):
        def load(g):
            ib = jnp.bitwise_and(g, 7)
            pltpu.async_copy(psrc.at[:, pl.ds(e0_of(g), K)], idx3.at[ib], semi.at[ib])
        def wait_load(g):
            ib = jnp.bitwise_and(g, 7)
            pltpu.make_async_copy(psrc.at[:, pl.ds(0, K)], idx3.at[ib], semi.at[ib]).wait()
        def gather_start(g):
            b = jnp.bitwise_and(g, 3)
            ib = jnp.bitwise_and(g, 7)
            pltpu.async_copy(gtab.at[idx3.at[ib, 0]], grows2.at[b], semg.at[b])
        def wait_gather(g):
            b = jnp.bitwise_and(g, 3)
            pltpu.make_async_copy(gtab.at[pl.ds(0, K)], grows2.at[b], semg.at[b]).wait()
        def scatter_start(g):
            b = jnp.bitwise_and(g, 3)
            ib = jnp.bitwise_and(g, 7)
            pltpu.async_copy(grows2.at[b], acc.at[idx3.at[ib, 1]], sems.at[b], add=True)
            if with_deg:
                pltpu.async_copy(dub, deg2d.at[idx3.at[ib, 1]], sems.at[b], add=True)
        def wait_scatter(g):
            b = jnp.bitwise_and(g, 3)
            pltpu.make_async_copy(grows2.at[b], acc.at[pl.ds(0, K)], sems.at[b]).wait()
            if with_deg:
                pltpu.make_async_copy(dub, deg2d.at[pl.ds(0, K)], sems.at[b]).wait()
        def scale(g):
            b = jnp.bitwise_and(g, 3)
            ib = jnp.bitwise_and(g, 7)
            @plsc.parallel_loop(0, K // 16, unroll=K // 16)
            def blk(t):
                j0 = t * 16
                vv = plsc.bitcast(idx3[ib, 2, pl.ds(j0, 16)], jnp.float32)
                for jj in range(16):
                    v = jnp.full((16,), vv[jj], jnp.float32)
                    for f in range(H // 16):
                        sl = pl.ds(f * 16, 16)
                        grows2[b, j0 + jj, sl] = grows2[b, j0 + jj, sl] * v

        for k in range(4):
            load(k)
        wait_load(0)
        gather_start(0)
        wait_load(1)
        gather_start(1)
        def body(g, _):
            @pl.when(g + 2 < nchunk)
            def _():
                @pl.when(g >= 2)
                def _():
                    wait_scatter(g + 2)
                wait_load(g + 2)
                gather_start(g + 2)
            wait_gather(g)
            scale(g)
            scatter_start(g)
            @pl.when(g + 4 < nchunk)
            def _():
                load(g + 4)
            return 0
        lax.fori_loop(0, nchunk, body, 0)
        for k in range(4):
            wait_scatter(k)

    def chunk_off(g):
        return jnp.where(g < CH, s * (CH * K) + g * K, EBASE + s * K)

    run_pipeline(chunk_off, ep_hbm, xw_hbm.at[c], True)
    plsc.subcore_barrier()

    def norm_q(q, _):
        r = r0 + q * NQ
        pltpu.sync_copy(acc.at[pl.ds(r, NQ)], nbuf)
        pltpu.sync_copy(zbuf, acc.at[pl.ds(r, NQ)])
        pltpu.sync_copy(base_hbm.at[c, pl.ds(r, NQ)], bbuf)
        pltpu.sync_copy(deg2d.at[pl.ds(r, NQ)], degbuf)
        def nrow(j, _):
            dv = degbuf[j, pl.ds(0, 16)]
            ivv = jnp.where(dv > 0.0, 1.0 / dv, 0.0)
            iv = jnp.full((16,), ivv[0], jnp.float32)
            for f in range(H // 16):
                sl = pl.ds(f * 16, 16)
                hbuf[j, sl] = nbuf[j, sl] * iv + bbuf[j, sl]
            return 0
        lax.fori_loop(0, NQ, nrow, 0)
        pltpu.sync_copy(hbuf, h_out.at[c, pl.ds(r, NQ)])
        return 0
    lax.fori_loop(0, RT // NQ, norm_q, 0)
    plsc.subcore_barrier()

    def motif_body(i, _):
        run_pipeline(lambda g: i * NNZ + chunk_off(g), mp_hbm, h_out.at[c], False)
        plsc.subcore_barrier()
        pltpu.sync_copy(acc.at[pl.ds(r0, RT)], s_out.at[i, c, pl.ds(r0, RT)])
        def zacc2(q, __):
            pltpu.sync_copy(zbuf, acc.at[pl.ds(r0 + q * NQ, NQ)])
            return 0
        lax.fori_loop(0, RT // NQ, zacc2, 0)
        plsc.subcore_barrier()
        return 0
    lax.fori_loop(0, NM, motif_body, 0)


def _sc_sparse(xw_pair, base_pair, epack, mpack):
    mesh = plsc.VectorSubcoreMesh(core_axis_name="c", subcore_axis_name="s")
    f = pl.kernel(
        _sc_body,
        out_type=[
            jax.ShapeDtypeStruct((2, NP, H), jnp.float32),
            jax.ShapeDtypeStruct((NM, 2, NP, H), jnp.float32),
        ],
        mesh=mesh,
        compiler_params=pltpu.CompilerParams(use_tc_tiling_on_sc=False,
                                             needs_layout_passes=False),
        scratch_types=[
            pltpu.VMEM((8, 3, K), jnp.int32),
            pltpu.VMEM((4, K, H), jnp.float32),
            pltpu.VMEM((NQ, H), jnp.float32),
            pltpu.VMEM((NQ, H), jnp.float32),
            pltpu.VMEM((NQ, H), jnp.float32),
            pltpu.VMEM((NQ, H), jnp.float32),
            pltpu.VMEM((K, 16), jnp.float32),
            pltpu.VMEM((NQ, 16), jnp.float32),
            pltpu.VMEM_SHARED((NP, H), jnp.float32),
            pltpu.VMEM_SHARED((NP, 16), jnp.float32),
            pltpu.SemaphoreType.DMA((8,)),
            pltpu.SemaphoreType.DMA((4,)),
            pltpu.SemaphoreType.DMA((4,)),
        ],
    )
    return f(xw_pair, base_pair, epack, mpack)


def _build_combined_weights(wa, motif_w):
    blocks = motif_w.reshape(NM, NM, C, D)
    vc = jnp.zeros((NM + 1, C, 2 * NM * D), jnp.float32)
    for i in range(1, NM + 1):
        for j in range(NM + 1):
            if j == i:
                continue
            jj = j if j < i else j - 1
            vc = vc.at[j, :, (i - 1) * D : i * D].set(blocks[i - 1, jj])
        vc = vc.at[i, :, NM * D + (i - 1) * D : NM * D + i * D].set(wa)
    return vc


def kernel(x, edge_weight, motif_val, W_conv, root, bias, wa, ba, motif_w, motif_b, edge_index, motif_idx):
    xw_pair, base_pair = _dense_a(x, W_conv, root, bias)

    epack = jnp.stack([
        edge_index[0], edge_index[1],
        lax.bitcast_convert_type(edge_weight, jnp.int32),
    ])
    mpack = jnp.stack([
        motif_idx[:, 1].reshape(-1), motif_idx[:, 0].reshape(-1),
        lax.bitcast_convert_type(motif_val.reshape(-1), jnp.int32),
    ])
    h_pair, s_all = _sc_sparse(xw_pair, base_pair, epack, mpack)

    vc = _build_combined_weights(wa, motif_w)
    cb = motif_b.reshape(1, NM * D)
    mb = jnp.tile(ba, NM).reshape(1, NM * D)
    return _attn(h_pair, s_all, vc, cb, mb)


# speedup vs baseline: 1.2264x; 1.0001x over previous
import functools
import jax
import jax.numpy as jnp
from jax import lax
from jax.experimental import pallas as pl
from jax.experimental.pallas import tpu as pltpu
from jax.experimental.pallas import tpu_sc as plsc

N = 10000
E = 320000
C = 128
D = 32
NM = 13
NNZ = 320000
H = 64
RB = 1000
NRB = N // RB


def _dense_a_kernel(x_ref, w_ref, r_ref, b_ref, xw_ref, base_ref):
    xb = x_ref[...]
    xw_ref[0] = jnp.dot(xb, w_ref[0], preferred_element_type=jnp.float32)
    base_ref[0] = jnp.dot(xb, r_ref[0], preferred_element_type=jnp.float32) + b_ref[0]


def _dense_a(x, W_conv, root, bias):
    w_split = W_conv.reshape(C, 2, H).transpose(1, 0, 2)
    r_split = root.reshape(C, 2, H).transpose(1, 0, 2)
    b_split = bias.reshape(2, 1, H)
    return pl.pallas_call(
        _dense_a_kernel,
        grid=(2, NRB),
        in_specs=[
            pl.BlockSpec((RB, C), lambda c, i: (i, 0)),
            pl.BlockSpec((1, C, H), lambda c, i: (c, 0, 0)),
            pl.BlockSpec((1, C, H), lambda c, i: (c, 0, 0)),
            pl.BlockSpec((1, 1, H), lambda c, i: (c, 0, 0)),
        ],
        out_specs=[
            pl.BlockSpec((1, RB, H), lambda c, i: (c, i, 0)),
            pl.BlockSpec((1, RB, H), lambda c, i: (c, i, 0)),
        ],
        out_shape=[
            jax.ShapeDtypeStruct((2, NP, H), jnp.float32),
            jax.ShapeDtypeStruct((2, NP, H), jnp.float32),
        ],
    )(x, w_split, r_split, b_split)


def _attn_kernel(h_ref, s_ref, vc_ref, cb_ref, mb_ref, o_ref):
    acc = jnp.zeros((RB, 2 * NM * D), jnp.float32)
    for m in range(NM + 1):
        if m == 0:
            rm = jnp.concatenate([h_ref[0], h_ref[1]], axis=1)
        else:
            rm = jnp.concatenate([s_ref[m - 1, 0], s_ref[m - 1, 1]], axis=1)
        acc = acc + jnp.dot(rm, vc_ref[m], preferred_element_type=jnp.float32)
    zc = acc[:, : NM * D] + cb_ref[...]
    zm = acc[:, NM * D :] + mb_ref[...]
    g = (lax.broadcasted_iota(jnp.int32, (NM * D, NM), 0) // D
         == lax.broadcasted_iota(jnp.int32, (NM * D, NM), 1)).astype(jnp.float32)
    logits = jnp.dot(zc * zm, g, preferred_element_type=jnp.float32)
    att = jax.nn.sigmoid(logits)
    att_e = jnp.dot(att, g.T, preferred_element_type=jnp.float32)
    o_ref[...] = att_e * (zm - zc)


def _attn(h_pair, s_all, vc, cb, mb):
    return pl.pallas_call(
        _attn_kernel,
        grid=(NRB,),
        in_specs=[
            pl.BlockSpec((2, RB, H), lambda i: (0, i, 0)),
            pl.BlockSpec((NM, 2, RB, H), lambda i: (0, 0, i, 0)),
            pl.BlockSpec((NM + 1, C, 2 * NM * D), lambda i: (0, 0, 0)),
            pl.BlockSpec((1, NM * D), lambda i: (0, 0)),
            pl.BlockSpec((1, NM * D), lambda i: (0, 0)),
        ],
        out_specs=pl.BlockSpec((RB, NM * D), lambda i: (i, 0)),
        out_shape=jax.ShapeDtypeStruct((N, NM * D), jnp.float32),
    )(h_pair, s_all, vc, cb, mb)


NS = 16
NP = 10240
RT = NP // NS
K = 128
CH = E // (NS * K)
EBASE = NS * CH * K
NQ = 32


def _sc_body(xw_hbm, base_hbm, ep_hbm, mp_hbm,
             h_out, s_out,
             idx3, grows2, nbuf, bbuf, hbuf, zbuf, dub, degbuf,
             acc, deg2d, semi, semg, sems):
    c = lax.axis_index("c")
    s = lax.axis_index("s")
    r0 = s * RT
    nchunk = jnp.where(s < 4, CH + 1, CH)

    z16 = jnp.zeros((16,), jnp.float32)
    dconst = jnp.where(jnp.arange(16, dtype=jnp.int32) == 0, 1.0, 0.0).astype(jnp.float32)

    def zrow(j, _):
        for f in range(H // 16):
            zbuf[j, pl.ds(f * 16, 16)] = z16
        degbuf[j, pl.ds(0, 16)] = z16
        return 0
    lax.fori_loop(0, NQ, zrow, 0)
    def drow(j, _):
        dub[j, pl.ds(0, 16)] = dconst
        return 0
    lax.fori_loop(0, K, drow, 0)
    def zacc(q, _):
        pltpu.sync_copy(zbuf, acc.at[pl.ds(r0 + q * NQ, NQ)])
        pltpu.sync_copy(degbuf, deg2d.at[pl.ds(r0 + q * NQ, NQ)])
        return 0
    lax.fori_loop(0, RT // NQ, zacc, 0)
    plsc.subcore_barrier()

    def run_pipeline(e0_of, psrc, gtab, with_deg):
        def load(g):
            ib = jnp.bitwise_and(g, 15)
            pltpu.async_copy(psrc.at[:, pl.ds(e0_of(g), K)], idx3.at[ib], semi.at[ib])
        def wait_load(g):
            ib = jnp.bitwise_and(g, 15)
            pltpu.make_async_copy(psrc.at[:, pl.ds(0, K)], idx3.at[ib], semi.at[ib]).wait()
        def gather_start(g):
            b = lax.rem(g, 5)
            ib = jnp.bitwise_and(g, 15)
            pltpu.async_copy(gtab.at[idx3.at[ib, 0]], grows2.at[b], semg.at[b])
        def wait_gather(g):
            b = lax.rem(g, 5)
            pltpu.make_async_copy(gtab.at[pl.ds(0, K)], grows2.at[b], semg.at[b]).wait()
        def scatter_start(g):
            b = lax.rem(g, 5)
            ib = jnp.bitwise_and(g, 15)
            pltpu.async_copy(grows2.at[b], acc.at[idx3.at[ib, 1]], sems.at[b], add=True)
            if with_deg:
                pltpu.async_copy(dub, deg2d.at[idx3.at[ib, 1]], sems.at[b], add=True)
        def wait_scatter(g):
            b = lax.rem(g, 5)
            pltpu.make_async_copy(grows2.at[b], acc.at[pl.ds(0, K)], sems.at[b]).wait()
            if with_deg:
                pltpu.make_async_copy(dub, deg2d.at[pl.ds(0, K)], sems.at[b]).wait()
        def scale(g):
            b = lax.rem(g, 5)
            ib = jnp.bitwise_and(g, 15)
            @plsc.parallel_loop(0, K // 16, unroll=K // 16)
            def blk(t):
                j0 = t * 16
                vv = plsc.bitcast(idx3[ib, 2, pl.ds(j0, 16)], jnp.float32)
                for jj in range(16):
                    v = jnp.full((16,), vv[jj], jnp.float32)
                    for f in range(H // 16):
                        sl = pl.ds(f * 16, 16)
                        grows2[b, j0 + jj, sl] = grows2[b, j0 + jj, sl] * v

        for k in range(5):
            load(k)
        for k in range(3):
            wait_load(k)
            gather_start(k)
        def body(g, _):
            @pl.when(g + 3 < nchunk)
            def _():
                @pl.when(g >= 2)
                def _():
                    wait_scatter(g + 3)
                wait_load(g + 3)
                gather_start(g + 3)
            wait_gather(g)
            scale(g)
            scatter_start(g)
            @pl.when(g + 5 < nchunk)
            def _():
                load(g + 5)
            return 0
        lax.fori_loop(0, nchunk, body, 0)
        for k in range(5):
            wait_scatter(k)

    def chunk_off(g):
        return jnp.where(g < CH, s * (CH * K) + g * K, EBASE + s * K)

    run_pipeline(chunk_off, ep_hbm, xw_hbm.at[c], True)
    plsc.subcore_barrier()

    def norm_q(q, _):
        r = r0 + q * NQ
        pltpu.sync_copy(acc.at[pl.ds(r, NQ)], nbuf)
        pltpu.sync_copy(zbuf, acc.at[pl.ds(r, NQ)])
        pltpu.sync_copy(base_hbm.at[c, pl.ds(r, NQ)], bbuf)
        pltpu.sync_copy(deg2d.at[pl.ds(r, NQ)], degbuf)
        def nrow(j, _):
            dv = degbuf[j, pl.ds(0, 16)]
            ivv = jnp.where(dv > 0.0, 1.0 / dv, 0.0)
            iv = jnp.full((16,), ivv[0], jnp.float32)
            for f in range(H // 16):
                sl = pl.ds(f * 16, 16)
                hbuf[j, sl] = nbuf[j, sl] * iv + bbuf[j, sl]
            return 0
        lax.fori_loop(0, NQ, nrow, 0)
        pltpu.sync_copy(hbuf, h_out.at[c, pl.ds(r, NQ)])
        return 0
    lax.fori_loop(0, RT // NQ, norm_q, 0)
    plsc.subcore_barrier()

    def motif_body(i, _):
        run_pipeline(lambda g: i * NNZ + chunk_off(g), mp_hbm, h_out.at[c], False)
        plsc.subcore_barrier()
        pltpu.sync_copy(acc.at[pl.ds(r0, RT)], s_out.at[i, c, pl.ds(r0, RT)])
        def zacc2(q, __):
            pltpu.sync_copy(zbuf, acc.at[pl.ds(r0 + q * NQ, NQ)])
            return 0
        lax.fori_loop(0, RT // NQ, zacc2, 0)
        plsc.subcore_barrier()
        return 0
    lax.fori_loop(0, NM, motif_body, 0)


def _sc_sparse(xw_pair, base_pair, epack, mpack):
    mesh = plsc.VectorSubcoreMesh(core_axis_name="c", subcore_axis_name="s")
    f = pl.kernel(
        _sc_body,
        out_type=[
            jax.ShapeDtypeStruct((2, NP, H), jnp.float32),
            jax.ShapeDtypeStruct((NM, 2, NP, H), jnp.float32),
        ],
        mesh=mesh,
        compiler_params=pltpu.CompilerParams(use_tc_tiling_on_sc=False,
                                             needs_layout_passes=False),
        scratch_types=[
            pltpu.VMEM((16, 3, K), jnp.int32),
            pltpu.VMEM((5, K, H), jnp.float32),
            pltpu.VMEM((NQ, H), jnp.float32),
            pltpu.VMEM((NQ, H), jnp.float32),
            pltpu.VMEM((NQ, H), jnp.float32),
            pltpu.VMEM((NQ, H), jnp.float32),
            pltpu.VMEM((K, 16), jnp.float32),
            pltpu.VMEM((NQ, 16), jnp.float32),
            pltpu.VMEM_SHARED((NP, H), jnp.float32),
            pltpu.VMEM_SHARED((NP, 16), jnp.float32),
            pltpu.SemaphoreType.DMA((16,)),
            pltpu.SemaphoreType.DMA((5,)),
            pltpu.SemaphoreType.DMA((5,)),
        ],
    )
    return f(xw_pair, base_pair, epack, mpack)


def _build_combined_weights(wa, motif_w):
    blocks = motif_w.reshape(NM, NM, C, D)
    vc = jnp.zeros((NM + 1, C, 2 * NM * D), jnp.float32)
    for i in range(1, NM + 1):
        for j in range(NM + 1):
            if j == i:
                continue
            jj = j if j < i else j - 1
            vc = vc.at[j, :, (i - 1) * D : i * D].set(blocks[i - 1, jj])
        vc = vc.at[i, :, NM * D + (i - 1) * D : NM * D + i * D].set(wa)
    return vc


def kernel(x, edge_weight, motif_val, W_conv, root, bias, wa, ba, motif_w, motif_b, edge_index, motif_idx):
    xw_pair, base_pair = _dense_a(x, W_conv, root, bias)

    epack = jnp.stack([
        edge_index[0], edge_index[1],
        lax.bitcast_convert_type(edge_weight, jnp.int32),
    ])
    mpack = jnp.stack([
        motif_idx[:, 1].reshape(-1), motif_idx[:, 0].reshape(-1),
        lax.bitcast_convert_type(motif_val.reshape(-1), jnp.int32),
    ])
    h_pair, s_all = _sc_sparse(xw_pair, base_pair, epack, mpack)

    vc = _build_combined_weights(wa, motif_w)
    cb = motif_b.reshape(1, NM * D)
    mb = jnp.tile(ba, NM).reshape(1, NM * D)
    return _attn(h_pair, s_all, vc, cb, mb)
